# Initial kernel scaffold; baseline (speedup 1.0000x reference)
import jax
import jax.numpy as jnp
from jax.experimental import pallas as pl


def kernel(edge_index, adj_values, features, W1, b1, W2, b2):
    raise NotImplementedError("write your pallas kernel here")


# speedup vs baseline: 4.5312x; 4.5312x over previous
import functools

import jax
import jax.numpy as jnp
from jax import lax
from jax.experimental import pallas as pl
from jax.experimental.pallas import tpu as pltpu
from jax.experimental.pallas import tpu_sc as plsc

N = 10000
E = 320000
D = 128

NUM_CORES = 2
NUM_SUBCORES = 16
NUM_TILES = NUM_CORES * NUM_SUBCORES
CHUNK = 128
CHUNKS_PER_TILE = 79
E_PAD = NUM_TILES * CHUNKS_PER_TILE * CHUNK
N_PAD = 10240
ROWS_PER_TILE = N_PAD // NUM_SUBCORES


def _sc_scatter(row3, col3, vals3, feat, out, row_v, col_v, vals_v, rows_v,
                acc, sem):
    cid = lax.axis_index("c")
    sid = lax.axis_index("s")
    wid = cid * NUM_SUBCORES + sid

    pltpu.sync_copy(row3.at[wid], row_v)
    pltpu.sync_copy(col3.at[wid], col_v)
    pltpu.sync_copy(vals3.at[wid], vals_v)

    def zero_body(i, _):
        for q in range(8):
            rows_v[i, pl.ds(q * 16, 16)] = jnp.zeros((16,), jnp.float32)
        return 0

    lax.fori_loop(0, CHUNK, zero_body, 0)

    rbase = sid * ROWS_PER_TILE
    for t in range(ROWS_PER_TILE // CHUNK):
        pltpu.sync_copy(rows_v, acc.at[pl.ds(rbase + t * CHUNK, CHUNK)])
    plsc.subcore_barrier()

    def chunk_body(t, _):
        pltpu.async_copy(feat.at[col_v.at[t]], rows_v, sem).wait()

        def scale_group(g, _):
            vv = vals_v[t, pl.ds(g * 16, 16)]
            base = g * 16
            for j in range(16):
                vj = lax.gather(
                    vv, jnp.full((16, 1), j, jnp.int32),
                    lax.GatherDimensionNumbers(
                        offset_dims=(), collapsed_slice_dims=(0,),
                        start_index_map=(0,)),
                    slice_sizes=(1,),
                    mode=lax.GatherScatterMode.PROMISE_IN_BOUNDS)
                for q in range(8):
                    sl = pl.ds(q * 16, 16)
                    rows_v[base + j, sl] = rows_v[base + j, sl] * vj
            return 0

        lax.fori_loop(0, CHUNK // 16, scale_group, 0)

        pltpu.sync_copy(rows_v, acc.at[row_v.at[t]], add=True)
        return 0

    lax.fori_loop(0, CHUNKS_PER_TILE, chunk_body, 0)
    plsc.subcore_barrier()

    pltpu.sync_copy(acc.at[pl.ds(rbase, ROWS_PER_TILE)],
                    out.at[cid, pl.ds(rbase, ROWS_PER_TILE)])


_sc_kernel = functools.partial(
    pl.kernel,
    out_type=jax.ShapeDtypeStruct((NUM_CORES, N_PAD, D), jnp.float32),
    mesh=plsc.VectorSubcoreMesh(core_axis_name="c", subcore_axis_name="s"),
    scratch_types=[
        pltpu.VMEM((CHUNKS_PER_TILE, CHUNK), jnp.int32),
        pltpu.VMEM((CHUNKS_PER_TILE, CHUNK), jnp.int32),
        pltpu.VMEM((CHUNKS_PER_TILE, CHUNK), jnp.float32),
        pltpu.VMEM((CHUNK, D), jnp.float32),
        pltpu.VMEM_SHARED((N_PAD, D), jnp.float32),
        pltpu.SemaphoreType.DMA,
    ],
)(_sc_scatter)


def _tc_dense(f_ref, h0_ref, h1_ref, w1_ref, w2_ref, b1_ref, b2_ref, o_ref):
    h = h0_ref[...] + h1_ref[...]
    f = f_ref[...]
    a = f + h
    m = f * h
    dims = (((1,), (1,)), ((), ()))
    y = lax.dot_general(a, w1_ref[...], dims,
                        preferred_element_type=jnp.float32)
    y += lax.dot_general(m, w2_ref[...], dims,
                         preferred_element_type=jnp.float32)
    y += b1_ref[...] + b2_ref[...]
    o_ref[...] = jnp.where(y >= 0, y, 0.01 * y)


BLOCK_ROWS = 400


def _tc_kernel(f, h0, h1, W1, W2, b1, b2):
    grid = (N // BLOCK_ROWS,)
    row_spec = pl.BlockSpec((BLOCK_ROWS, D), lambda i: (i, 0))
    full_spec = pl.BlockSpec((D, D), lambda i: (0, 0))
    bias_spec = pl.BlockSpec((1, D), lambda i: (0, 0))
    return pl.pallas_call(
        _tc_dense,
        grid=grid,
        in_specs=[row_spec, row_spec, row_spec, full_spec, full_spec,
                  bias_spec, bias_spec],
        out_specs=row_spec,
        out_shape=jax.ShapeDtypeStruct((N, D), jnp.float32),
    )(f, h0, h1, W1, W2, b1, b2)


@jax.jit
def kernel(edge_index, adj_values, features, W1, b1, W2, b2):
    row = edge_index[0].astype(jnp.int32)
    col = edge_index[1].astype(jnp.int32)
    vals = adj_values.astype(jnp.float32)

    pad = E_PAD - E
    row3 = jnp.concatenate([row, jnp.zeros((pad,), jnp.int32)]).reshape(
        NUM_TILES, CHUNKS_PER_TILE, CHUNK)
    col3 = jnp.concatenate([col, jnp.zeros((pad,), jnp.int32)]).reshape(
        NUM_TILES, CHUNKS_PER_TILE, CHUNK)
    vals3 = jnp.concatenate([vals, jnp.zeros((pad,), jnp.float32)]).reshape(
        NUM_TILES, CHUNKS_PER_TILE, CHUNK)

    partials = _sc_kernel(row3, col3, vals3, features)

    return _tc_kernel(features, partials[0], partials[1], W1, W2,
                      b1.reshape(1, D), b2.reshape(1, D))
